# baseline (speedup 1.0000x reference)
import functools

import jax
import jax.numpy as jnp
from jax import lax
from jax.experimental import pallas as pl
from jax.experimental.pallas import tpu as pltpu
from jax.experimental.pallas import tpu_sc as plsc

_NC, _NS = 2, 16
_NW = _NC * _NS
_CHUNK = 128


def _pairs_body(xT_ref, eye_ref, out_ref):
    x = xT_ref[...]
    eye = eye_ref[...]
    half = x.shape[1] // 2
    c0 = (((0,), (0,)), ((), ()))
    top = lax.dot_general(x[:, 0:half], eye, c0,
                          preferred_element_type=jnp.float32)
    bot = lax.dot_general(x[:, half:], eye, c0,
                          preferred_element_type=jnp.float32)
    out_ref[...] = jnp.concatenate([top, bot], axis=1)


_PAIR_U = 32768


def _make_pairs(tabT, D):
    n = tabT.shape[1]
    U = _PAIR_U
    grid = (n + U - 1) // U
    return pl.pallas_call(
        _pairs_body,
        grid=(grid,),
        in_specs=[pl.BlockSpec((D, U), lambda i: (0, i)),
                  pl.BlockSpec((D, D), lambda i: (0, 0))],
        out_specs=pl.BlockSpec((U // 2, 2 * D), lambda i: (i, 0)),
        out_shape=jax.ShapeDtypeStruct((grid * U // 2, 2 * D), jnp.float32),
        compiler_params=pltpu.CompilerParams(
            dimension_semantics=("arbitrary",)),
    )(tabT, jnp.eye(D, dtype=jnp.float32))


def _sc_gather(pairs, pidx2d, B):
    b_per_w = B // _NW
    n_chunks = b_per_w // _CHUNK
    mesh = plsc.VectorSubcoreMesh(core_axis_name="c", subcore_axis_name="s")

    @functools.partial(
        pl.kernel,
        mesh=mesh,
        out_type=jax.ShapeDtypeStruct((B, 128), jnp.float32),
        scratch_types=[
            pltpu.VMEM((n_chunks, _CHUNK), jnp.int32),
            pltpu.VMEM((b_per_w, 128), jnp.float32),
            pltpu.SemaphoreType.DMA,
            pltpu.SemaphoreType.DMA,
        ],
    )
    def k(tab_hbm, pidx_hbm, out_hbm, idx_v, rows_v, gsem, wsem):
        wid = lax.axis_index("s") * _NC + lax.axis_index("c")
        base = wid * b_per_w
        row0 = wid * n_chunks

        pltpu.sync_copy(pidx_hbm.at[pl.ds(row0, n_chunks)], idx_v)
        gathers = []
        for j in range(n_chunks):
            gathers.append(pltpu.async_copy(
                tab_hbm.at[idx_v.at[j]],
                rows_v.at[pl.ds(j * _CHUNK, _CHUNK)], gsem))
        writes = []
        for j in range(n_chunks):
            gathers[j].wait()
            writes.append(pltpu.async_copy(
                rows_v.at[pl.ds(j * _CHUNK, _CHUNK)],
                out_hbm.at[pl.ds(base + j * _CHUNK, _CHUNK)], wsem))
        for w in writes:
            w.wait()

    return k(pairs, pidx2d)


def _mlp_body(uep, iep, upar, ipar, ufT, itfT,
              ufW1, ufb1, ufW2, ufb2, ifW1, ifb1, ifW2, ifb2,
              W0u, W0i, W0uf, W0if, b0, W1, b1, W2, b2, W3, b3,
              out_ref):
    zero = jnp.float32(0.0)
    contract0 = (((0,), (0,)), ((), ()))
    up = uep[...]
    ue = jnp.where(upar[...] > zero, up[:, 64:128], up[:, 0:64])
    ip = iep[...]
    ie = jnp.where(ipar[...] > zero, ip[:, 64:128], ip[:, 0:64])
    u = jnp.maximum(lax.dot_general(ufT[...], ufW1[...], contract0)
                    + ufb1[...], zero)
    u = jnp.maximum(u @ ufW2[...] + ufb2[...], zero)
    v = jnp.maximum(lax.dot_general(itfT[...], ifW1[...], contract0)
                    + ifb1[...], zero)
    v = jnp.maximum(v @ ifW2[...] + ifb2[...], zero)
    h = (ue @ W0u[...] + ie @ W0i[...]
         + u @ W0uf[...] + v @ W0if[...] + b0[...])
    h = jnp.maximum(h, zero)
    h = jnp.maximum(h @ W1[...] + b1[...], zero)
    h = jnp.maximum(h @ W2[...] + b2[...], zero)
    out_ref[...] = h @ W3[...] + b3[...]


def kernel(user_ids, item_ids, user_features, item_features, params):
    p = params
    B = user_ids.shape[0]
    D = p['user_table'].shape[1]
    eps = 1e-5

    item_pairs = _make_pairs(p['item_table'].T, D)
    user_pairs = _make_pairs(p['user_table'].T, D)

    U = _PAIR_U

    def pair_idx(ids):
        return (ids // U) * (U // 2) + (ids % (U // 2))

    def pair_half(ids):
        return ((ids % U) // (U // 2)).astype(jnp.float32)

    upar = pair_half(user_ids).reshape(B, 1)
    ipar = pair_half(item_ids).reshape(B, 1)
    item_emb = _sc_gather(item_pairs,
                          pair_idx(item_ids).reshape(B // _CHUNK, _CHUNK), B)
    user_emb = _sc_gather(user_pairs,
                          pair_idx(user_ids).reshape(B // _CHUNK, _CHUNK), B)

    s0 = p['g0'] / jnp.sqrt(p['v0'] + eps)
    t0 = p['be0'] - p['m0'] * s0
    s1 = p['g1'] / jnp.sqrt(p['v1'] + eps)
    t1 = p['be1'] - p['m1'] * s1
    s2 = p['g2'] / jnp.sqrt(p['v2'] + eps)
    t2 = p['be2'] - p['m2'] * s2
    W1f = s0[:, None] * p['W1']
    b1f = p['b1'] + t0 @ p['W1']
    W2f = s1[:, None] * p['W2']
    b2f = p['b2'] + t1 @ p['W2']
    W3f = s2[:, None] * p['W3']
    b3f = p['b3'] + t2 @ p['W3']

    W0 = p['W0']
    W0u, W0i, W0uf, W0if = W0[0:D], W0[D:2 * D], W0[2 * D:3 * D], W0[3 * D:4 * D]

    BB = 2048
    row2d = lambda a: a.reshape(1, -1)
    full = lambda a: pl.BlockSpec(a.shape, lambda i: (0, 0))
    weights = [p['uf_W1'], row2d(p['uf_b1']), p['uf_W2'], row2d(p['uf_b2']),
               p['if_W1'], row2d(p['if_b1']), p['if_W2'], row2d(p['if_b2']),
               W0u, W0i, W0uf, W0if, row2d(p['b0']),
               W1f, row2d(b1f), W2f, row2d(b2f), W3f, row2d(b3f)]

    out = pl.pallas_call(
        _mlp_body,
        grid=(B // BB,),
        in_specs=[
            pl.BlockSpec((BB, 2 * D), lambda i: (i, 0)),
            pl.BlockSpec((BB, 2 * D), lambda i: (i, 0)),
            pl.BlockSpec((BB, 1), lambda i: (i, 0)),
            pl.BlockSpec((BB, 1), lambda i: (i, 0)),
            pl.BlockSpec((user_features.shape[1], BB), lambda i: (0, i)),
            pl.BlockSpec((item_features.shape[1], BB), lambda i: (0, i)),
        ] + [full(w) for w in weights],
        out_specs=pl.BlockSpec((BB, 1), lambda i: (i, 0)),
        out_shape=jax.ShapeDtypeStruct((B, 1), jnp.float32),
    )(user_emb, item_emb, upar, ipar,
      user_features.T, item_features.T, *weights)
    return out[:, 0]


# speedup vs baseline: 1.1343x; 1.1343x over previous
import functools

import jax
import jax.numpy as jnp
from jax import lax
from jax.experimental import pallas as pl
from jax.experimental.pallas import tpu as pltpu
from jax.experimental.pallas import tpu_sc as plsc

_NC, _NS = 2, 16
_NW = _NC * _NS
_CHUNK = 128


def _pairs_body(xT_ref, eye_ref, out_ref):
    x = xT_ref[...].astype(jnp.bfloat16)
    eye = eye_ref[...].astype(jnp.bfloat16)
    half = x.shape[1] // 2
    c0 = (((0,), (0,)), ((), ()))
    top = lax.dot_general(x[:, 0:half], eye, c0,
                          preferred_element_type=jnp.float32)
    bot = lax.dot_general(x[:, half:], eye, c0,
                          preferred_element_type=jnp.float32)
    out_ref[...] = jnp.concatenate([top, bot], axis=1)


_PAIR_U = 32768


def _make_pairs(tabT, D):
    n = tabT.shape[1]
    U = _PAIR_U
    grid = (n + U - 1) // U
    return pl.pallas_call(
        _pairs_body,
        grid=(grid,),
        in_specs=[pl.BlockSpec((D, U), lambda i: (0, i)),
                  pl.BlockSpec((D, D), lambda i: (0, 0))],
        out_specs=pl.BlockSpec((U // 2, 2 * D), lambda i: (i, 0)),
        out_shape=jax.ShapeDtypeStruct((grid * U // 2, 2 * D), jnp.float32),
        compiler_params=pltpu.CompilerParams(
            dimension_semantics=("arbitrary",)),
    )(tabT, jnp.eye(D, dtype=jnp.float32))


def _sc_gather(pairs, pidx2d, B):
    b_per_w = B // _NW
    n_chunks = b_per_w // _CHUNK
    mesh = plsc.VectorSubcoreMesh(core_axis_name="c", subcore_axis_name="s")

    @functools.partial(
        pl.kernel,
        mesh=mesh,
        out_type=jax.ShapeDtypeStruct((B, 128), jnp.float32),
        scratch_types=[
            pltpu.VMEM((n_chunks, _CHUNK), jnp.int32),
            pltpu.VMEM((b_per_w, 128), jnp.float32),
            pltpu.SemaphoreType.DMA,
            pltpu.SemaphoreType.DMA,
        ],
    )
    def k(tab_hbm, pidx_hbm, out_hbm, idx_v, rows_v, gsem, wsem):
        wid = lax.axis_index("s") * _NC + lax.axis_index("c")
        base = wid * b_per_w
        row0 = wid * n_chunks

        pltpu.sync_copy(pidx_hbm.at[pl.ds(row0, n_chunks)], idx_v)
        gathers = []
        for j in range(n_chunks):
            gathers.append(pltpu.async_copy(
                tab_hbm.at[idx_v.at[j]],
                rows_v.at[pl.ds(j * _CHUNK, _CHUNK)], gsem))
        writes = []
        for j in range(n_chunks):
            gathers[j].wait()
            writes.append(pltpu.async_copy(
                rows_v.at[pl.ds(j * _CHUNK, _CHUNK)],
                out_hbm.at[pl.ds(base + j * _CHUNK, _CHUNK)], wsem))
        for w in writes:
            w.wait()

    return k(pairs, pidx2d)


def _mlp_body(uep, iep, upar, ipar, ufT, itfT,
              ufW1, ufb1, ufW2, ufb2, ifW1, ifb1, ifW2, ifb2,
              W0u, W0i, W0uf, W0if, b0, W1, b1, W2, b2, W3, b3,
              out_ref):
    zero = jnp.float32(0.0)
    contract0 = (((0,), (0,)), ((), ()))
    up = uep[...]
    ue = jnp.where(upar[...] > zero, up[:, 64:128], up[:, 0:64])
    ip = iep[...]
    ie = jnp.where(ipar[...] > zero, ip[:, 64:128], ip[:, 0:64])
    u = jnp.maximum(lax.dot_general(ufT[...], ufW1[...], contract0)
                    + ufb1[...], zero)
    u = jnp.maximum(u @ ufW2[...] + ufb2[...], zero)
    v = jnp.maximum(lax.dot_general(itfT[...], ifW1[...], contract0)
                    + ifb1[...], zero)
    v = jnp.maximum(v @ ifW2[...] + ifb2[...], zero)
    h = (ue @ W0u[...] + ie @ W0i[...]
         + u @ W0uf[...] + v @ W0if[...] + b0[...])
    h = jnp.maximum(h, zero)
    h = jnp.maximum(h @ W1[...] + b1[...], zero)
    h = jnp.maximum(h @ W2[...] + b2[...], zero)
    out_ref[...] = h @ W3[...] + b3[...]


def kernel(user_ids, item_ids, user_features, item_features, params):
    p = params
    B = user_ids.shape[0]
    D = p['user_table'].shape[1]
    eps = 1e-5

    item_pairs = _make_pairs(p['item_table'].T, D)
    user_pairs = _make_pairs(p['user_table'].T, D)

    U = _PAIR_U

    def pair_idx(ids):
        return (ids // U) * (U // 2) + (ids % (U // 2))

    def pair_half(ids):
        return ((ids % U) // (U // 2)).astype(jnp.float32)

    upar = pair_half(user_ids).reshape(B, 1)
    ipar = pair_half(item_ids).reshape(B, 1)
    item_emb = _sc_gather(item_pairs,
                          pair_idx(item_ids).reshape(B // _CHUNK, _CHUNK), B)
    user_emb = _sc_gather(user_pairs,
                          pair_idx(user_ids).reshape(B // _CHUNK, _CHUNK), B)

    s0 = p['g0'] / jnp.sqrt(p['v0'] + eps)
    t0 = p['be0'] - p['m0'] * s0
    s1 = p['g1'] / jnp.sqrt(p['v1'] + eps)
    t1 = p['be1'] - p['m1'] * s1
    s2 = p['g2'] / jnp.sqrt(p['v2'] + eps)
    t2 = p['be2'] - p['m2'] * s2
    W1f = s0[:, None] * p['W1']
    b1f = p['b1'] + t0 @ p['W1']
    W2f = s1[:, None] * p['W2']
    b2f = p['b2'] + t1 @ p['W2']
    W3f = s2[:, None] * p['W3']
    b3f = p['b3'] + t2 @ p['W3']

    W0 = p['W0']
    W0u, W0i, W0uf, W0if = W0[0:D], W0[D:2 * D], W0[2 * D:3 * D], W0[3 * D:4 * D]

    BB = 2048
    row2d = lambda a: a.reshape(1, -1)
    full = lambda a: pl.BlockSpec(a.shape, lambda i: (0, 0))
    weights = [p['uf_W1'], row2d(p['uf_b1']), p['uf_W2'], row2d(p['uf_b2']),
               p['if_W1'], row2d(p['if_b1']), p['if_W2'], row2d(p['if_b2']),
               W0u, W0i, W0uf, W0if, row2d(p['b0']),
               W1f, row2d(b1f), W2f, row2d(b2f), W3f, row2d(b3f)]

    out = pl.pallas_call(
        _mlp_body,
        grid=(B // BB,),
        in_specs=[
            pl.BlockSpec((BB, 2 * D), lambda i: (i, 0)),
            pl.BlockSpec((BB, 2 * D), lambda i: (i, 0)),
            pl.BlockSpec((BB, 1), lambda i: (i, 0)),
            pl.BlockSpec((BB, 1), lambda i: (i, 0)),
            pl.BlockSpec((user_features.shape[1], BB), lambda i: (0, i)),
            pl.BlockSpec((item_features.shape[1], BB), lambda i: (0, i)),
        ] + [full(w) for w in weights],
        out_specs=pl.BlockSpec((BB, 1), lambda i: (i, 0)),
        out_shape=jax.ShapeDtypeStruct((B, 1), jnp.float32),
    )(user_emb, item_emb, upar, ipar,
      user_features.T, item_features.T, *weights)
    return out[:, 0]


# speedup vs baseline: 1.1603x; 1.0229x over previous
import functools

import jax
import jax.numpy as jnp
from jax import lax
from jax.experimental import pallas as pl
from jax.experimental.pallas import tpu as pltpu
from jax.experimental.pallas import tpu_sc as plsc

_NC, _NS = 2, 16
_NW = _NC * _NS
_CHUNK = 128


def _pairs_body(xT_ref, eye_ref, out_ref):
    x = xT_ref[...].astype(jnp.bfloat16)
    eye = eye_ref[...].astype(jnp.bfloat16)
    half = x.shape[1] // 2
    c0 = (((0,), (0,)), ((), ()))
    top = lax.dot_general(x[:, 0:half], eye, c0,
                          preferred_element_type=jnp.float32)
    bot = lax.dot_general(x[:, half:], eye, c0,
                          preferred_element_type=jnp.float32)
    out_ref[...] = jnp.concatenate([top, bot], axis=1)


_PAIR_U = 32768


def _make_pairs(tabT, D):
    n = tabT.shape[1]
    U = _PAIR_U
    grid = (n + U - 1) // U
    return pl.pallas_call(
        _pairs_body,
        grid=(grid,),
        in_specs=[pl.BlockSpec((D, U), lambda i: (0, i)),
                  pl.BlockSpec((D, D), lambda i: (0, 0))],
        out_specs=pl.BlockSpec((U // 2, 2 * D), lambda i: (i, 0)),
        out_shape=jax.ShapeDtypeStruct((grid * U // 2, 2 * D), jnp.float32),
        compiler_params=pltpu.CompilerParams(
            dimension_semantics=("arbitrary",)),
    )(tabT, jnp.eye(D, dtype=jnp.float32))


def _sc_gather(pairs, pidx2d, B):
    b_per_w = B // _NW
    n_chunks = b_per_w // _CHUNK
    mesh = plsc.VectorSubcoreMesh(core_axis_name="c", subcore_axis_name="s")

    @functools.partial(
        pl.kernel,
        mesh=mesh,
        out_type=jax.ShapeDtypeStruct((B, 128), jnp.float32),
        scratch_types=[
            pltpu.VMEM((n_chunks, _CHUNK), jnp.int32),
            pltpu.VMEM((b_per_w, 128), jnp.float32),
            pltpu.SemaphoreType.DMA,
            pltpu.SemaphoreType.DMA,
        ],
    )
    def k(tab_hbm, pidx_hbm, out_hbm, idx_v, rows_v, gsem, wsem):
        wid = lax.axis_index("s") * _NC + lax.axis_index("c")
        base = wid * b_per_w
        row0 = wid * n_chunks

        pltpu.sync_copy(pidx_hbm.at[pl.ds(row0, n_chunks)], idx_v)
        gathers = []
        for j in range(n_chunks):
            gathers.append(pltpu.async_copy(
                tab_hbm.at[idx_v.at[j]],
                rows_v.at[pl.ds(j * _CHUNK, _CHUNK)], gsem))
        writes = []
        for j in range(n_chunks):
            gathers[j].wait()
            writes.append(pltpu.async_copy(
                rows_v.at[pl.ds(j * _CHUNK, _CHUNK)],
                out_hbm.at[pl.ds(base + j * _CHUNK, _CHUNK)], wsem))
        for w in writes:
            w.wait()

    return k(pairs, pidx2d)


def _mlp_body(uep, iep, upar, ipar, ufT, itfT,
              ufW1, ufb1, ufW2, ufb2, ifW1, ifb1, ifW2, ifb2,
              W0u, W0i, W0uf, W0if, b0, s0, t0, W1, b1, s1, t1,
              W2, b2, s2, t2, W3, b3,
              out_ref):
    zero = jnp.float32(0.0)
    contract0 = (((0,), (0,)), ((), ()))
    up = uep[...]
    ue = jnp.where(upar[...] > zero, up[:, 64:128], up[:, 0:64])
    ip = iep[...]
    ie = jnp.where(ipar[...] > zero, ip[:, 64:128], ip[:, 0:64])
    u = jnp.maximum(lax.dot_general(ufT[...], ufW1[...], contract0)
                    + ufb1[...], zero)
    u = jnp.maximum(u @ ufW2[...] + ufb2[...], zero)
    v = jnp.maximum(lax.dot_general(itfT[...], ifW1[...], contract0)
                    + ifb1[...], zero)
    v = jnp.maximum(v @ ifW2[...] + ifb2[...], zero)
    h = (ue @ W0u[...] + ie @ W0i[...]
         + u @ W0uf[...] + v @ W0if[...] + b0[...])
    h = jnp.maximum(h, zero) * s0[...] + t0[...]
    h = jnp.maximum(h @ W1[...] + b1[...], zero) * s1[...] + t1[...]
    h = jnp.maximum(h @ W2[...] + b2[...], zero) * s2[...] + t2[...]
    out_ref[...] = h @ W3[...] + b3[...]


def kernel(user_ids, item_ids, user_features, item_features, params):
    p = params
    B = user_ids.shape[0]
    D = p['user_table'].shape[1]
    eps = 1e-5

    item_pairs = _make_pairs(p['item_table'].T, D)
    user_pairs = _make_pairs(p['user_table'].T, D)

    U = _PAIR_U

    def pair_idx(ids):
        return (ids // U) * (U // 2) + (ids % (U // 2))

    def pair_half(ids):
        return ((ids % U) // (U // 2)).astype(jnp.float32)

    upar = pair_half(user_ids).reshape(B, 1)
    ipar = pair_half(item_ids).reshape(B, 1)
    item_emb = _sc_gather(item_pairs,
                          pair_idx(item_ids).reshape(B // _CHUNK, _CHUNK), B)
    user_emb = _sc_gather(user_pairs,
                          pair_idx(user_ids).reshape(B // _CHUNK, _CHUNK), B)

    s0 = p['g0'] / jnp.sqrt(p['v0'] + eps)
    t0 = p['be0'] - p['m0'] * s0
    s1 = p['g1'] / jnp.sqrt(p['v1'] + eps)
    t1 = p['be1'] - p['m1'] * s1
    s2 = p['g2'] / jnp.sqrt(p['v2'] + eps)
    t2 = p['be2'] - p['m2'] * s2

    W0 = p['W0']
    W0u, W0i, W0uf, W0if = W0[0:D], W0[D:2 * D], W0[2 * D:3 * D], W0[3 * D:4 * D]

    BB = 4096
    row2d = lambda a: a.reshape(1, -1)
    full = lambda a: pl.BlockSpec(a.shape, lambda i: (0, 0))
    weights = [p['uf_W1'], row2d(p['uf_b1']), p['uf_W2'], row2d(p['uf_b2']),
               p['if_W1'], row2d(p['if_b1']), p['if_W2'], row2d(p['if_b2']),
               W0u, W0i, W0uf, W0if, row2d(p['b0']),
               row2d(s0), row2d(t0),
               p['W1'], row2d(p['b1']), row2d(s1), row2d(t1),
               p['W2'], row2d(p['b2']), row2d(s2), row2d(t2),
               p['W3'], row2d(p['b3'])]

    out = pl.pallas_call(
        _mlp_body,
        grid=(B // BB,),
        in_specs=[
            pl.BlockSpec((BB, 2 * D), lambda i: (i, 0)),
            pl.BlockSpec((BB, 2 * D), lambda i: (i, 0)),
            pl.BlockSpec((BB, 1), lambda i: (i, 0)),
            pl.BlockSpec((BB, 1), lambda i: (i, 0)),
            pl.BlockSpec((user_features.shape[1], BB), lambda i: (0, i)),
            pl.BlockSpec((item_features.shape[1], BB), lambda i: (0, i)),
        ] + [full(w) for w in weights],
        out_specs=pl.BlockSpec((BB, 1), lambda i: (i, 0)),
        out_shape=jax.ShapeDtypeStruct((B, 1), jnp.float32),
    )(user_emb, item_emb, upar, ipar,
      user_features.T, item_features.T, *weights)
    return out[:, 0]
